# baseline (speedup 1.0000x reference)
import functools

import jax
import jax.numpy as jnp
from jax import lax
from jax.experimental import pallas as pl
from jax.experimental.pallas import tpu as pltpu
from jax.experimental.pallas import tpu_sc as plsc

N = 10000
E = 320000
B = 128
NW = 32
EC = 80
EPAD = NW * EC * 128
NSEG = 10240
ROWS_PT = NSEG // 16
NPAD = 10112
NBLK = NPAD // 128
RWIN = 264
NXR = 10384
FMAX_SEG = 257


def _make_agg(F, mask_edges, nh=1):
    mesh = plsc.VectorSubcoreMesh(core_axis_name="c", subcore_axis_name="s")

    @functools.partial(
        pl.kernel,
        mesh=mesh,
        compiler_params=pltpu.CompilerParams(use_tc_tiling_on_sc=False,
                                             needs_layout_passes=False),
        out_type=jax.ShapeDtypeStruct((nh, 2, NSEG, F), jnp.float32),
        scratch_types=[
            pltpu.VMEM((EC, 128), jnp.int32),
            pltpu.VMEM((EC, 128), jnp.int32),
            pltpu.VMEM((NSEG,), jnp.int32),
            pltpu.VMEM((4, 128, F), jnp.float32),
            pltpu.VMEM_SHARED((NSEG, F), jnp.float32),
            pltpu.SemaphoreType.DMA,
            pltpu.SemaphoreType.DMA,
            pltpu.SemaphoreType.DMA,
            pltpu.SemaphoreType.DMA,
        ],
    )
    def agg(*refs):
        h_hbms = refs[:nh]
        (src_hbm, dst_hbm, act_hbm, zero_hbm, out_hbm,
         src_v, dst_v, act_v, rows_v, acc, sem0, sem1, sem2, sem3) = refs[nh:]
        c = lax.axis_index("c")
        s = lax.axis_index("s")
        wid = s * 2 + c
        pltpu.sync_copy(src_hbm.at[wid], src_v)
        pltpu.sync_copy(dst_hbm.at[wid], dst_v)
        if mask_edges:
            pltpu.sync_copy(act_hbm, act_v)

            def mask_body(j, _):
                for l in range(8):
                    s16 = src_v[j, pl.ds(l * 16, 16)]
                    d16 = dst_v[j, pl.ds(l * 16, 16)]
                    a_s = plsc.load_gather(act_v, [s16])
                    a_d = plsc.load_gather(act_v, [d16])
                    ok = (a_s + a_d) == 2
                    dst_v[j, pl.ds(l * 16, 16)] = jnp.where(ok, d16, N)
                return 0

            lax.fori_loop(0, EC, mask_body, 0)

        sems = (sem0, sem1, sem2, sem3)
        for ph in range(nh):
            h_hbm = h_hbms[ph]
            pltpu.sync_copy(zero_hbm.at[pl.ds(s * ROWS_PT, ROWS_PT)],
                            acc.at[pl.ds(s * ROWS_PT, ROWS_PT)])
            plsc.subcore_barrier()

            def gstart(j, b):
                pltpu.async_copy(h_hbm.at[src_v.at[j]], rows_v.at[b], sems[b])

            def gwait(b):
                pltpu.make_async_copy(h_hbm.at[src_v.at[0]], rows_v.at[b],
                                      sems[b]).wait()

            def scat(j, b):
                pltpu.sync_copy(rows_v.at[b], acc.at[dst_v.at[j]], add=True)

            for b in range(4):
                gstart(b, b)

            def chunk4(jj, _):
                j0 = jj * 4
                for b in range(4):
                    gwait(b)
                    scat(j0 + b, b)

                    @pl.when(j0 + b + 4 < EC)
                    def _():
                        gstart(j0 + b + 4, b)
                return 0

            lax.fori_loop(0, EC // 4, chunk4, 0)
            plsc.subcore_barrier()
            pltpu.sync_copy(acc.at[pl.ds(s * ROWS_PT, ROWS_PT)],
                            out_hbm.at[ph, c, pl.ds(s * ROWS_PT, ROWS_PT)])

    return agg


@functools.lru_cache(maxsize=None)
def _get_agg(F, mask_edges, nh=1):
    return _make_agg(F, mask_edges, nh)


def _agg128_nomask(h, src, dst, act, zero64):
    return _get_agg(64, False, 2)(h[:, :64], h[:, 64:], src, dst, act, zero64)


def _agg32_nomask(*a):
    return _get_agg(32, False)(*a)


def _agg64_mask(*a):
    return _get_agg(64, True)(*a)


def _make_conv_post(needs_proj, with_score, split_agg=False):
    def body(agg_ref, x_ref, wr_ref, wroot_ref, b_ref, g_ref, be_ref,
             mf_ref, w_ref, y_ref, s_ref=None):
        if split_agg:
            a = jnp.concatenate(
                [agg_ref[0, 0, :N, :] + agg_ref[0, 1, :N, :],
                 agg_ref[1, 0, :N, :] + agg_ref[1, 1, :N, :]], axis=1)
        else:
            a = agg_ref[0, 0, :N, :] + agg_ref[0, 1, :N, :]
        if needs_proj:
            a = jnp.dot(a, wr_ref[...], preferred_element_type=jnp.float32)
        pre = a + b_ref[...] + jnp.dot(x_ref[...], wroot_ref[...],
                                       preferred_element_type=jnp.float32)
        mf = mf_ref[...]
        n = jnp.sum(mf)
        mu = jnp.sum(pre * mf, axis=0, keepdims=True) / n
        var = jnp.sum(((pre - mu) * mf) ** 2, axis=0, keepdims=True) / n
        y = g_ref[...] * (pre - mu) * lax.rsqrt(var + 1e-5) + be_ref[...]
        y = jnp.maximum(y, 0.0)
        y_ref[...] = y
        if with_score:
            w = w_ref[...]
            nw = jnp.sqrt(jnp.sum(w * w))
            s_ref[...] = jnp.tanh(jnp.dot(y, w, preferred_element_type=jnp.float32) / nw)

    def run(agg, x, wr, wroot, bias, g, be, mf, w):
        fout = wroot.shape[1]
        outs = [jax.ShapeDtypeStruct((N, fout), jnp.float32)]
        if with_score:
            outs.append(jax.ShapeDtypeStruct((N, 1), jnp.float32))
        r = pl.pallas_call(body, out_shape=outs)(
            agg, x, wr, wroot, bias, g, be, mf, w)
        return r if with_score else (r[0], None)

    return run


_conv_post_first = _make_conv_post(True, False, split_agg=True)
_conv_post_score = _make_conv_post(True, True)


def _seg_setup_body(b_ref, cnt_ref, s0_ref):
    jj = lax.broadcasted_iota(jnp.int32, (128, 128), 1)
    cnt = jnp.zeros((1, 128), jnp.int32)
    s0 = jnp.zeros((1, 128), jnp.int32)
    for r in range(NBLK):
        bb = b_ref[pl.ds(r * 128, 128), :]
        cnt = cnt + jnp.sum((bb == jj).astype(jnp.int32), axis=0, keepdims=True)
        s0 = s0 + jnp.sum((bb < jj).astype(jnp.int32), axis=0, keepdims=True)
    cnt_ref[...] = cnt
    s0_ref[...] = s0


def _seg_setup(bcol):
    return pl.pallas_call(
        _seg_setup_body,
        out_shape=[jax.ShapeDtypeStruct((1, 128), jnp.int32),
                   jax.ShapeDtypeStruct((1, 128), jnp.int32)],
    )(bcol)


def _make_pool(ratio):
    def body(y_ref, scol_ref, bcol_ref, pcol_ref, spad_ref, bpad_ref,
             ppad_ref, cntr_ref, cntc_ref, s0_ref,
             nx_ref, act_ref, nbat_ref, npos_ref, kr_ref, kc_ref, o_ref,
             kf_ref):
        kv = jnp.ceil(ratio * cntr_ref[...].astype(jnp.float32)).astype(jnp.int32)
        kcol = jnp.ceil(ratio * cntc_ref[...].astype(jnp.float32)).astype(jnp.int32)
        kr_ref[...] = kv
        kc_ref[...] = kcol
        kf_ref[...] = jnp.maximum(kcol.astype(jnp.float32), 1.0)
        jj = lax.broadcasted_iota(jnp.int32, (128, 128), 1)
        for r in range(NBLK):
            rows = pl.ds(r * 128, 128)
            bb = bcol_ref[rows, :]
            sb = scol_ref[rows, :]
            pb = pcol_ref[rows, :]
            racc = jnp.zeros((128, 1), jnp.int32)
            for w in range(5):
                sw = spad_ref[pl.ds(r + w, 1), :]
                bw = bpad_ref[pl.ds(r + w, 1), :]
                pw = ppad_ref[pl.ds(r + w, 1), :]
                same = (bw == bb) & (bb < B)
                better = (sw > sb) | ((sw == sb) & (pw < pb))
                racc = racc + jnp.sum((same & better).astype(jnp.int32),
                                      axis=1, keepdims=True)
            kk = jnp.sum(jnp.where(jj == bb, kv, 0), axis=1, keepdims=True)
            ks = jnp.sum(jnp.where(jj < bb, kv, 0), axis=1, keepdims=True)
            keep = (bb < B) & (racc < kk)
            act_ref[rows, :] = keep.astype(jnp.int32)
            nbat_ref[rows, :] = jnp.where(keep, bb, B)
            npos_ref[rows, :] = ks + racc
        nbat_ref[pl.ds(NPAD, NXR - NPAD), :] = jnp.full((NXR - NPAD, 1), B,
                                                        jnp.int32)
        nx_ref[pl.ds(0, N), :] = y_ref[...] * scol_ref[pl.ds(0, N), :]

        def one(b, _):
            st = s0_ref[0, b]
            st8 = (st // 8) * 8
            win = nx_ref[pl.ds(st8, RWIN), :]
            bwin = nbat_ref[pl.ds(st8, RWIN), :]
            m = bwin == b
            neg = jnp.float32(-jnp.inf)
            gmp = jnp.max(jnp.where(m, win, neg), axis=0, keepdims=True)
            cb = kf_ref[pl.ds(b, 1), :]
            gap = jnp.sum(jnp.where(m, win, 0.0), axis=0, keepdims=True) / cb
            o_ref[pl.ds(b, 1), 0:64] = gmp
            o_ref[pl.ds(b, 1), 64:128] = gap
            return 0

        lax.fori_loop(0, B, one, 0)

    def run(y, scol, bcol, pcol, spad, bpad, ppad, cnt_row, cnt_col, s0):
        outs = [jax.ShapeDtypeStruct((NXR, 64), jnp.float32),
                jax.ShapeDtypeStruct((NPAD, 1), jnp.int32),
                jax.ShapeDtypeStruct((NXR, 1), jnp.int32),
                jax.ShapeDtypeStruct((NPAD, 1), jnp.int32),
                jax.ShapeDtypeStruct((1, 128), jnp.int32),
                jax.ShapeDtypeStruct((128, 1), jnp.int32),
                jax.ShapeDtypeStruct((B, 128), jnp.float32)]
        specs = [pl.BlockSpec(memory_space=pltpu.VMEM)] * 9 + [
            pl.BlockSpec(memory_space=pltpu.SMEM)]
        return pl.pallas_call(
            body, in_specs=specs, out_shape=outs,
            scratch_shapes=[pltpu.VMEM((128, 1), jnp.float32)],
        )(y, scol, bcol, pcol, spad, bpad, ppad, cnt_row, cnt_col, s0)

    return run


_pool_half = _make_pool(0.5)
_pool_03 = _make_pool(0.3)


def _head_body(x1, x2, x3, x4, w1, b1, w2, b2, w3, b3, o_ref):
    h = x4[...] + x3[...] + x2[...] + x1[...]
    h = jnp.maximum(jnp.dot(h, w1[...], preferred_element_type=jnp.float32)
                    + b1[...], 0.0)
    h = jnp.maximum(jnp.dot(h, w2[...], preferred_element_type=jnp.float32)
                    + b2[...], 0.0)
    z = jnp.dot(h, w3[...], preferred_element_type=jnp.float32) + b3[...]
    zm = z - jnp.max(z, axis=1, keepdims=True)
    o_ref[...] = zm - jnp.log(jnp.sum(jnp.exp(zm), axis=1, keepdims=True))


def _head(x1, x2, x3, x4, p):
    return pl.pallas_call(
        _head_body,
        out_shape=jax.ShapeDtypeStruct((B, 16), jnp.float32),
    )(x1, x2, x3, x4,
      p['l1_W'], p['l1_b'].reshape(1, -1),
      p['l2_W'], p['l2_b'].reshape(1, -1),
      p['l3_W'], p['l3_b'].reshape(1, -1))


def _pad_col(v, rows, fill):
    return jnp.pad(v, ((0, rows - v.shape[0]), (0, 0)), constant_values=fill)


def kernel(x, edge_index, batch, params):
    p = params
    i32 = jnp.int32
    src = jnp.concatenate([edge_index[0].astype(i32),
                           jnp.zeros((EPAD - E,), i32)]).reshape(NW, EC, 128)
    dst = jnp.concatenate([edge_index[1].astype(i32),
                           jnp.full((EPAD - E,), N, i32)]).reshape(NW, EC, 128)
    zero64 = jnp.zeros((NSEG, 64), jnp.float32)
    zero32 = jnp.zeros((NSEG, 32), jnp.float32)
    act_all = jnp.ones((NSEG,), i32)
    ones_mf = jnp.ones((N, 1), jnp.float32)
    dummy_w = jnp.zeros((32, 1), jnp.float32)

    bcol = _pad_col(batch.astype(i32)[:, None], NPAD, B)
    pcol = jnp.arange(NPAD, dtype=i32)[:, None]
    cnt_row, s0 = _seg_setup(bcol)
    cnt_col = cnt_row.reshape(128, 1)

    agg = _agg128_nomask(x, src, dst, act_all, zero64)
    y, _ = _conv_post_first(agg, x, p['c1_Wr'], p['c1_Wroot'],
                            p['c1_b'].reshape(1, -1), p['bn1_g'].reshape(1, -1),
                            p['bn1_b'].reshape(1, -1), ones_mf, dummy_w)
    agg = _agg32_nomask(y, src, dst, act_all, zero32)
    y, scol = _conv_post_score(agg, y, p['c1b_Wr'], p['c1b_Wroot'],
                               p['c1b_b'].reshape(1, -1),
                               p['bn1b_g'].reshape(1, -1),
                               p['bn1b_b'].reshape(1, -1), ones_mf,
                               p['p1_w'].reshape(-1, 1))

    xs = []
    stages = [('c2', 'bn2', 'p2_w'),
              ('c3', 'bn3', 'p3_w'),
              ('c4', 'bn4', 'p4_w'),
              (None, None, None)]
    pools = [_pool_half, _pool_half, _pool_half, _pool_03]
    for li in range(4):
        scp = _pad_col(scol, NPAD, 0.0)
        spad = jnp.pad(scp.reshape(NBLK, 128), ((2, 2), (0, 0)))
        bpad = jnp.pad(bcol.reshape(NBLK, 128), ((2, 2), (0, 0)),
                       constant_values=B)
        ppad = jnp.pad(pcol.reshape(NBLK, 128), ((2, 2), (0, 0)))
        nx, actc, nbat, npos, k_row, k_col, xcat = pools[li](
            y, scp, bcol, pcol, spad, bpad, ppad, cnt_row, cnt_col, s0)
        bcol, pcol, cnt_row, cnt_col = nbat[:NPAD], npos, k_row, k_col
        xs.append(xcat)
        cname, bname, wname = stages[li]
        if cname is None:
            break
        act_ext = jnp.concatenate([actc[:N, 0], jnp.zeros((NSEG - N,), i32)])
        mf = actc[:N].astype(jnp.float32)
        agg = _agg64_mask(nx[:N], src, dst, act_ext, zero64)
        y, scol = _conv_post_score(agg, nx[:N], p[cname + '_Wr'],
                                   p[cname + '_Wroot'],
                                   p[cname + '_b'].reshape(1, -1),
                                   p[bname + '_g'].reshape(1, -1),
                                   p[bname + '_b'].reshape(1, -1), mf,
                                   p[wname].reshape(-1, 1))

    return _head(xs[0], xs[1], xs[2], xs[3], p)


# speedup vs baseline: 1.0245x; 1.0245x over previous
import functools

import jax
import jax.numpy as jnp
from jax import lax
from jax.experimental import pallas as pl
from jax.experimental.pallas import tpu as pltpu
from jax.experimental.pallas import tpu_sc as plsc

N = 10000
E = 320000
B = 128
NW = 32
EC = 80
EPAD = NW * EC * 128
NSEG = 10240
ROWS_PT = NSEG // 16
NPAD = 10112
NBLK = NPAD // 128
RWIN = 264
NXR = 10384
FMAX_SEG = 257


def _make_agg(F, mask_edges, nh=1):
    mesh = plsc.VectorSubcoreMesh(core_axis_name="c", subcore_axis_name="s")

    @functools.partial(
        pl.kernel,
        mesh=mesh,
        compiler_params=pltpu.CompilerParams(use_tc_tiling_on_sc=False,
                                             needs_layout_passes=False),
        out_type=jax.ShapeDtypeStruct((nh, 2, NSEG, F), jnp.float32),
        scratch_types=[
            pltpu.VMEM((EC, 128), jnp.int32),
            pltpu.VMEM((EC, 128), jnp.int32),
            pltpu.VMEM((NSEG,), jnp.int32),
            pltpu.VMEM((4, 128, F), jnp.float32),
            pltpu.VMEM_SHARED((NSEG, F), jnp.float32),
            pltpu.SemaphoreType.DMA,
            pltpu.SemaphoreType.DMA,
            pltpu.SemaphoreType.DMA,
            pltpu.SemaphoreType.DMA,
        ],
    )
    def agg(*refs):
        h_hbms = refs[:nh]
        (src_hbm, dst_hbm, act_hbm, zero_hbm, out_hbm,
         src_v, dst_v, act_v, rows_v, acc, sem0, sem1, sem2, sem3) = refs[nh:]
        c = lax.axis_index("c")
        s = lax.axis_index("s")
        wid = s * 2 + c
        pltpu.sync_copy(src_hbm.at[wid], src_v)
        pltpu.sync_copy(dst_hbm.at[wid], dst_v)
        if mask_edges:
            pltpu.sync_copy(act_hbm, act_v)

            def mask_body(j, _):
                for l in range(8):
                    s16 = src_v[j, pl.ds(l * 16, 16)]
                    d16 = dst_v[j, pl.ds(l * 16, 16)]
                    a_s = plsc.load_gather(act_v, [s16])
                    a_d = plsc.load_gather(act_v, [d16])
                    ok = (a_s + a_d) == 2
                    dst_v[j, pl.ds(l * 16, 16)] = jnp.where(ok, d16, N)
                return 0

            lax.fori_loop(0, EC, mask_body, 0)

        sems = (sem0, sem1, sem2, sem3)
        for ph in range(nh):
            h_hbm = h_hbms[ph]
            pltpu.sync_copy(zero_hbm.at[pl.ds(s * ROWS_PT, ROWS_PT)],
                            acc.at[pl.ds(s * ROWS_PT, ROWS_PT)])
            plsc.subcore_barrier()

            def gstart(j, b):
                pltpu.async_copy(h_hbm.at[src_v.at[j]], rows_v.at[b], sems[b])

            def gwait(b):
                pltpu.make_async_copy(h_hbm.at[src_v.at[0]], rows_v.at[b],
                                      sems[b]).wait()

            def scat(j, b):
                pltpu.sync_copy(rows_v.at[b], acc.at[dst_v.at[j]], add=True)

            for b in range(4):
                gstart(b, b)

            def chunk4(jj, _):
                j0 = jj * 4
                for b in range(4):
                    gwait(b)
                    scat(j0 + b, b)

                    @pl.when(j0 + b + 4 < EC)
                    def _():
                        gstart(j0 + b + 4, b)
                return 0

            lax.fori_loop(0, EC // 4, chunk4, 0)
            plsc.subcore_barrier()
            pltpu.sync_copy(acc.at[pl.ds(s * ROWS_PT, ROWS_PT)],
                            out_hbm.at[ph, c, pl.ds(s * ROWS_PT, ROWS_PT)])

    return agg


@functools.lru_cache(maxsize=None)
def _get_agg(F, mask_edges, nh=1):
    return _make_agg(F, mask_edges, nh)


def _agg128_nomask(h, src, dst, act, zero64):
    return _get_agg(64, False, 2)(h[:, :64], h[:, 64:], src, dst, act, zero64)


def _agg32_nomask(*a):
    return _get_agg(32, False)(*a)


def _agg64_mask(*a):
    return _get_agg(64, True)(*a)


def _make_conv_post(needs_proj, with_score, split_agg=False):
    def body(agg_ref, x_ref, wr_ref, wroot_ref, b_ref, g_ref, be_ref,
             mf_ref, w_ref, y_ref, s_ref=None):
        if split_agg:
            a = jnp.concatenate(
                [agg_ref[0, 0, :N, :] + agg_ref[0, 1, :N, :],
                 agg_ref[1, 0, :N, :] + agg_ref[1, 1, :N, :]], axis=1)
        else:
            a = agg_ref[0, 0, :N, :] + agg_ref[0, 1, :N, :]
        if needs_proj:
            a = jnp.dot(a, wr_ref[...], preferred_element_type=jnp.float32)
        pre = a + b_ref[...] + jnp.dot(x_ref[pl.ds(0, N), :], wroot_ref[...],
                                       preferred_element_type=jnp.float32)
        mf = mf_ref[...]
        n = jnp.sum(mf)
        mu = jnp.sum(pre * mf, axis=0, keepdims=True) / n
        var = jnp.sum(((pre - mu) * mf) ** 2, axis=0, keepdims=True) / n
        y = g_ref[...] * (pre - mu) * lax.rsqrt(var + 1e-5) + be_ref[...]
        y = jnp.maximum(y, 0.0)
        y_ref[...] = y
        if with_score:
            w = w_ref[...]
            nw = jnp.sqrt(jnp.sum(w * w))
            s_ref[...] = jnp.tanh(jnp.dot(y, w, preferred_element_type=jnp.float32) / nw)

    def run(agg, x, wr, wroot, bias, g, be, mf, w):
        fout = wroot.shape[1]
        outs = [jax.ShapeDtypeStruct((N, fout), jnp.float32)]
        if with_score:
            outs.append(jax.ShapeDtypeStruct((N, 1), jnp.float32))
        r = pl.pallas_call(body, out_shape=outs)(
            agg, x, wr, wroot, bias, g, be, mf, w)
        return r if with_score else (r[0], None)

    return run


_conv_post_first = _make_conv_post(True, False, split_agg=True)
_conv_post_score = _make_conv_post(True, True)


def _seg_setup_body(b_ref, cnt_ref, s0_ref):
    jj = lax.broadcasted_iota(jnp.int32, (128, 128), 1)
    cnt = jnp.zeros((1, 128), jnp.int32)
    s0 = jnp.zeros((1, 128), jnp.int32)
    for r in range(NBLK):
        bb = b_ref[pl.ds(r * 128, 128), :]
        cnt = cnt + jnp.sum((bb == jj).astype(jnp.int32), axis=0, keepdims=True)
        s0 = s0 + jnp.sum((bb < jj).astype(jnp.int32), axis=0, keepdims=True)
    cnt_ref[...] = cnt
    s0_ref[...] = s0


def _seg_setup(bcol):
    return pl.pallas_call(
        _seg_setup_body,
        out_shape=[jax.ShapeDtypeStruct((1, 128), jnp.int32),
                   jax.ShapeDtypeStruct((1, 128), jnp.int32)],
    )(bcol)


def _make_pool(ratio):
    def body(y_ref, scol_ref, bcol_ref, pcol_ref, spad_ref, bpad_ref,
             ppad_ref, cntr_ref, cntc_ref, s0_ref,
             nx_ref, act_ref, nbat_ref, npos_ref, kr_ref, kc_ref, o_ref,
             kf_ref):
        kv = jnp.ceil(ratio * cntr_ref[...].astype(jnp.float32)).astype(jnp.int32)
        kcol = jnp.ceil(ratio * cntc_ref[...].astype(jnp.float32)).astype(jnp.int32)
        kr_ref[...] = kv
        kc_ref[...] = kcol
        kf_ref[...] = jnp.maximum(kcol.astype(jnp.float32), 1.0)
        jj = lax.broadcasted_iota(jnp.int32, (128, 128), 1)
        for r in range(NBLK):
            rows = pl.ds(r * 128, 128)
            bb = bcol_ref[rows, :]
            sb = scol_ref[rows, :]
            pb = pcol_ref[rows, :]
            racc = jnp.zeros((128, 1), jnp.int32)
            for w in range(5):
                sw = spad_ref[pl.ds(r + w, 1), :]
                bw = bpad_ref[pl.ds(r + w, 1), :]
                pw = ppad_ref[pl.ds(r + w, 1), :]
                same = (bw == bb) & (bb < B)
                better = (sw > sb) | ((sw == sb) & (pw < pb))
                racc = racc + jnp.sum((same & better).astype(jnp.int32),
                                      axis=1, keepdims=True)
            kk = jnp.sum(jnp.where(jj == bb, kv, 0), axis=1, keepdims=True)
            ks = jnp.sum(jnp.where(jj < bb, kv, 0), axis=1, keepdims=True)
            keep = (bb < B) & (racc < kk)
            act_ref[rows, :] = keep.astype(jnp.int32)
            nbat_ref[rows, :] = jnp.where(keep, bb, B)
            npos_ref[rows, :] = ks + racc
        nbat_ref[pl.ds(NPAD, NXR - NPAD), :] = jnp.full((NXR - NPAD, 1), B,
                                                        jnp.int32)
        nx_ref[pl.ds(0, N), :] = y_ref[...] * scol_ref[pl.ds(0, N), :]

        def one(b, _):
            st = s0_ref[0, b]
            st8 = (st // 8) * 8
            win = nx_ref[pl.ds(st8, RWIN), :]
            bwin = nbat_ref[pl.ds(st8, RWIN), :]
            m = bwin == b
            neg = jnp.float32(-jnp.inf)
            gmp = jnp.max(jnp.where(m, win, neg), axis=0, keepdims=True)
            cb = kf_ref[pl.ds(b, 1), :]
            gap = jnp.sum(jnp.where(m, win, 0.0), axis=0, keepdims=True) / cb
            o_ref[pl.ds(b, 1), 0:64] = gmp
            o_ref[pl.ds(b, 1), 64:128] = gap
            return 0

        lax.fori_loop(0, B, one, 0)

    def run(y, scol, bcol, pcol, spad, bpad, ppad, cnt_row, cnt_col, s0):
        outs = [jax.ShapeDtypeStruct((NXR, 64), jnp.float32),
                jax.ShapeDtypeStruct((NPAD, 1), jnp.int32),
                jax.ShapeDtypeStruct((NXR, 1), jnp.int32),
                jax.ShapeDtypeStruct((NPAD, 1), jnp.int32),
                jax.ShapeDtypeStruct((1, 128), jnp.int32),
                jax.ShapeDtypeStruct((128, 1), jnp.int32),
                jax.ShapeDtypeStruct((B, 128), jnp.float32)]
        specs = [pl.BlockSpec(memory_space=pltpu.VMEM)] * 9 + [
            pl.BlockSpec(memory_space=pltpu.SMEM)]
        return pl.pallas_call(
            body, in_specs=specs, out_shape=outs,
            scratch_shapes=[pltpu.VMEM((128, 1), jnp.float32)],
        )(y, scol, bcol, pcol, spad, bpad, ppad, cnt_row, cnt_col, s0)

    return run


_pool_half = _make_pool(0.5)
_pool_03 = _make_pool(0.3)


def _head_body(x1, x2, x3, x4, w1, b1, w2, b2, w3, b3, o_ref):
    h = x4[...] + x3[...] + x2[...] + x1[...]
    h = jnp.maximum(jnp.dot(h, w1[...], preferred_element_type=jnp.float32)
                    + b1[...], 0.0)
    h = jnp.maximum(jnp.dot(h, w2[...], preferred_element_type=jnp.float32)
                    + b2[...], 0.0)
    z = jnp.dot(h, w3[...], preferred_element_type=jnp.float32) + b3[...]
    zm = z - jnp.max(z, axis=1, keepdims=True)
    o_ref[...] = zm - jnp.log(jnp.sum(jnp.exp(zm), axis=1, keepdims=True))


def _head(x1, x2, x3, x4, p):
    return pl.pallas_call(
        _head_body,
        out_shape=jax.ShapeDtypeStruct((B, 16), jnp.float32),
    )(x1, x2, x3, x4,
      p['l1_W'], p['l1_b'].reshape(1, -1),
      p['l2_W'], p['l2_b'].reshape(1, -1),
      p['l3_W'], p['l3_b'].reshape(1, -1))


def _pad_col(v, rows, fill):
    return jnp.pad(v, ((0, rows - v.shape[0]), (0, 0)), constant_values=fill)


def kernel(x, edge_index, batch, params):
    p = params
    i32 = jnp.int32
    src = jnp.concatenate([edge_index[0].astype(i32),
                           jnp.zeros((EPAD - E,), i32)]).reshape(NW, EC, 128)
    dst = jnp.concatenate([edge_index[1].astype(i32),
                           jnp.full((EPAD - E,), N, i32)]).reshape(NW, EC, 128)
    zero64 = jnp.zeros((NSEG, 64), jnp.float32)
    zero32 = jnp.zeros((NSEG, 32), jnp.float32)
    act_all = jnp.ones((NSEG,), i32)
    ones_mf = jnp.ones((N, 1), jnp.float32)
    dummy_w = jnp.zeros((32, 1), jnp.float32)

    bcol = _pad_col(batch.astype(i32)[:, None], NPAD, B)
    pcol = jnp.arange(NPAD, dtype=i32)[:, None]
    cnt_row, s0 = _seg_setup(bcol)
    cnt_col = cnt_row.reshape(128, 1)

    agg = _agg128_nomask(x, src, dst, act_all, zero64)
    y, _ = _conv_post_first(agg, x, p['c1_Wr'], p['c1_Wroot'],
                            p['c1_b'].reshape(1, -1), p['bn1_g'].reshape(1, -1),
                            p['bn1_b'].reshape(1, -1), ones_mf, dummy_w)
    agg = _agg32_nomask(y, src, dst, act_all, zero32)
    y, scol = _conv_post_score(agg, y, p['c1b_Wr'], p['c1b_Wroot'],
                               p['c1b_b'].reshape(1, -1),
                               p['bn1b_g'].reshape(1, -1),
                               p['bn1b_b'].reshape(1, -1), ones_mf,
                               p['p1_w'].reshape(-1, 1))

    xs = []
    stages = [('c2', 'bn2', 'p2_w'),
              ('c3', 'bn3', 'p3_w'),
              ('c4', 'bn4', 'p4_w'),
              (None, None, None)]
    pools = [_pool_half, _pool_half, _pool_half, _pool_03]
    for li in range(4):
        scp = _pad_col(scol, NPAD, 0.0)
        spad = jnp.pad(scp.reshape(NBLK, 128), ((2, 2), (0, 0)))
        bpad = jnp.pad(bcol.reshape(NBLK, 128), ((2, 2), (0, 0)),
                       constant_values=B)
        ppad = jnp.pad(pcol.reshape(NBLK, 128), ((2, 2), (0, 0)))
        nx, actc, nbat, npos, k_row, k_col, xcat = pools[li](
            y, scp, bcol, pcol, spad, bpad, ppad, cnt_row, cnt_col, s0)
        bcol, pcol, cnt_row, cnt_col = nbat[:NPAD], npos, k_row, k_col
        xs.append(xcat)
        cname, bname, wname = stages[li]
        if cname is None:
            break
        act_ext = jnp.concatenate([actc[:N, 0], jnp.zeros((NSEG - N,), i32)])
        mf = actc[:N].astype(jnp.float32)
        agg = _agg64_mask(nx, src, dst, act_ext, zero64)
        y, scol = _conv_post_score(agg, nx, p[cname + '_Wr'],
                                   p[cname + '_Wroot'],
                                   p[cname + '_b'].reshape(1, -1),
                                   p[bname + '_g'].reshape(1, -1),
                                   p[bname + '_b'].reshape(1, -1), mf,
                                   p[wname].reshape(-1, 1))

    return _head(xs[0], xs[1], xs[2], xs[3], p)


# speedup vs baseline: 1.0523x; 1.0272x over previous
import functools

import jax
import jax.numpy as jnp
from jax import lax
from jax.experimental import pallas as pl
from jax.experimental.pallas import tpu as pltpu
from jax.experimental.pallas import tpu_sc as plsc

N = 10000
E = 320000
B = 128
NW = 32
EC = 80
EPAD = NW * EC * 128
NSEG = 10240
ROWS_PT = NSEG // 16
NPAD = 10112
NBLK = NPAD // 128
RWIN = 264
NXR = 10384
FMAX_SEG = 257


def _make_agg(F, mask_edges, nh=1):
    mesh = plsc.VectorSubcoreMesh(core_axis_name="c", subcore_axis_name="s")

    @functools.partial(
        pl.kernel,
        mesh=mesh,
        compiler_params=pltpu.CompilerParams(use_tc_tiling_on_sc=False,
                                             needs_layout_passes=False),
        out_type=jax.ShapeDtypeStruct((nh, 2, NSEG, F), jnp.float32),
        scratch_types=[
            pltpu.VMEM((EC, 128), jnp.int32),
            pltpu.VMEM((EC, 128), jnp.int32),
            pltpu.VMEM((NSEG,), jnp.int32),
            pltpu.VMEM((4, 128, F), jnp.float32),
            pltpu.VMEM_SHARED((NSEG, F), jnp.float32),
            pltpu.SemaphoreType.DMA,
            pltpu.SemaphoreType.DMA,
            pltpu.SemaphoreType.DMA,
            pltpu.SemaphoreType.DMA,
        ],
    )
    def agg(*refs):
        h_hbms = refs[:nh]
        (src_hbm, dst_hbm, act_hbm, zero_hbm, out_hbm,
         src_v, dst_v, act_v, rows_v, acc, sem0, sem1, sem2, sem3) = refs[nh:]
        c = lax.axis_index("c")
        s = lax.axis_index("s")
        wid = s * 2 + c
        pltpu.sync_copy(src_hbm.at[wid], src_v)
        pltpu.sync_copy(dst_hbm.at[wid], dst_v)
        if mask_edges:
            pltpu.sync_copy(act_hbm, act_v)

            def mask_body(j, _):
                for l in range(8):
                    s16 = src_v[j, pl.ds(l * 16, 16)]
                    d16 = dst_v[j, pl.ds(l * 16, 16)]
                    a_s = plsc.load_gather(act_v, [s16])
                    a_d = plsc.load_gather(act_v, [d16])
                    ok = (a_s + a_d) == 2
                    dst_v[j, pl.ds(l * 16, 16)] = jnp.where(ok, d16, N)
                return 0

            lax.fori_loop(0, EC, mask_body, 0)

        sems = (sem0, sem1, sem2, sem3)
        for ph in range(nh):
            h_hbm = h_hbms[ph]
            pltpu.sync_copy(zero_hbm.at[pl.ds(s * ROWS_PT, ROWS_PT)],
                            acc.at[pl.ds(s * ROWS_PT, ROWS_PT)])
            plsc.subcore_barrier()

            def gstart(j, b):
                pltpu.async_copy(h_hbm.at[src_v.at[j]], rows_v.at[b], sems[b])

            def gwait(b):
                pltpu.make_async_copy(h_hbm.at[src_v.at[0]], rows_v.at[b],
                                      sems[b]).wait()

            def scat(j, b):
                pltpu.sync_copy(rows_v.at[b], acc.at[dst_v.at[j]], add=True)

            for b in range(4):
                gstart(b, b)

            def chunk4(jj, _):
                j0 = jj * 4
                for b in range(4):
                    gwait(b)
                    scat(j0 + b, b)

                    @pl.when(j0 + b + 4 < EC)
                    def _():
                        gstart(j0 + b + 4, b)
                return 0

            lax.fori_loop(0, EC // 4, chunk4, 0)
            plsc.subcore_barrier()
            pltpu.sync_copy(acc.at[pl.ds(s * ROWS_PT, ROWS_PT)],
                            out_hbm.at[ph, c, pl.ds(s * ROWS_PT, ROWS_PT)])

    return agg


@functools.lru_cache(maxsize=None)
def _get_agg(F, mask_edges, nh=1):
    return _make_agg(F, mask_edges, nh)


def _agg128_nomask(h, src, dst, act, zero64):
    f = _get_agg(64, False, 1)
    return jnp.concatenate(
        [f(h[:, :64], src, dst, act, zero64),
         f(h[:, 64:], src, dst, act, zero64)], axis=0)


def _agg32_nomask(*a):
    return _get_agg(32, False)(*a)


def _agg64_mask(*a):
    return _get_agg(64, True)(*a)


def _make_conv_post(needs_proj, with_score, split_agg=False):
    def body(agg_ref, x_ref, wr_ref, wroot_ref, b_ref, g_ref, be_ref,
             mf_ref, w_ref, y_ref, s_ref=None):
        if split_agg:
            a = jnp.concatenate(
                [agg_ref[0, 0, :N, :] + agg_ref[0, 1, :N, :],
                 agg_ref[1, 0, :N, :] + agg_ref[1, 1, :N, :]], axis=1)
        else:
            a = agg_ref[0, 0, :N, :] + agg_ref[0, 1, :N, :]
        if needs_proj:
            a = jnp.dot(a, wr_ref[...], preferred_element_type=jnp.float32)
        pre = a + b_ref[...] + jnp.dot(x_ref[pl.ds(0, N), :], wroot_ref[...],
                                       preferred_element_type=jnp.float32)
        mf = mf_ref[...]
        n = jnp.sum(mf)
        mu = jnp.sum(pre * mf, axis=0, keepdims=True) / n
        var = jnp.sum(((pre - mu) * mf) ** 2, axis=0, keepdims=True) / n
        y = g_ref[...] * (pre - mu) * lax.rsqrt(var + 1e-5) + be_ref[...]
        y = jnp.maximum(y, 0.0)
        y_ref[...] = y
        if with_score:
            w = w_ref[...]
            nw = jnp.sqrt(jnp.sum(w * w))
            s_ref[...] = jnp.tanh(jnp.dot(y, w, preferred_element_type=jnp.float32) / nw)

    def run(agg, x, wr, wroot, bias, g, be, mf, w):
        fout = wroot.shape[1]
        outs = [jax.ShapeDtypeStruct((N, fout), jnp.float32)]
        if with_score:
            outs.append(jax.ShapeDtypeStruct((N, 1), jnp.float32))
        r = pl.pallas_call(body, out_shape=outs)(
            agg, x, wr, wroot, bias, g, be, mf, w)
        return r if with_score else (r[0], None)

    return run


_conv_post_first = _make_conv_post(True, False, split_agg=True)
_conv_post_score = _make_conv_post(True, True)


def _seg_setup_body(b_ref, cnt_ref, s0_ref):
    jj = lax.broadcasted_iota(jnp.int32, (128, 128), 1)
    cnt = jnp.zeros((1, 128), jnp.int32)
    s0 = jnp.zeros((1, 128), jnp.int32)
    for r in range(NBLK):
        bb = b_ref[pl.ds(r * 128, 128), :]
        cnt = cnt + jnp.sum((bb == jj).astype(jnp.int32), axis=0, keepdims=True)
        s0 = s0 + jnp.sum((bb < jj).astype(jnp.int32), axis=0, keepdims=True)
    cnt_ref[...] = cnt
    s0_ref[...] = s0


def _seg_setup(bcol):
    return pl.pallas_call(
        _seg_setup_body,
        out_shape=[jax.ShapeDtypeStruct((1, 128), jnp.int32),
                   jax.ShapeDtypeStruct((1, 128), jnp.int32)],
    )(bcol)


def _make_pool(ratio):
    def body(y_ref, scol_ref, bcol_ref, pcol_ref, spad_ref, bpad_ref,
             ppad_ref, cntr_ref, cntc_ref, s0_ref,
             nx_ref, act_ref, nbat_ref, npos_ref, kr_ref, kc_ref, o_ref,
             kf_ref):
        kv = jnp.ceil(ratio * cntr_ref[...].astype(jnp.float32)).astype(jnp.int32)
        kcol = jnp.ceil(ratio * cntc_ref[...].astype(jnp.float32)).astype(jnp.int32)
        kr_ref[...] = kv
        kc_ref[...] = kcol
        kf_ref[...] = jnp.maximum(kcol.astype(jnp.float32), 1.0)
        jj = lax.broadcasted_iota(jnp.int32, (128, 128), 1)
        for r in range(NBLK):
            rows = pl.ds(r * 128, 128)
            bb = bcol_ref[rows, :]
            sb = scol_ref[rows, :]
            pb = pcol_ref[rows, :]
            racc = jnp.zeros((128, 1), jnp.int32)
            for w in range(5):
                sw = spad_ref[pl.ds(r + w, 1), :]
                bw = bpad_ref[pl.ds(r + w, 1), :]
                pw = ppad_ref[pl.ds(r + w, 1), :]
                same = (bw == bb) & (bb < B)
                better = (sw > sb) | ((sw == sb) & (pw < pb))
                racc = racc + jnp.sum((same & better).astype(jnp.int32),
                                      axis=1, keepdims=True)
            kk = jnp.sum(jnp.where(jj == bb, kv, 0), axis=1, keepdims=True)
            ks = jnp.sum(jnp.where(jj < bb, kv, 0), axis=1, keepdims=True)
            keep = (bb < B) & (racc < kk)
            act_ref[rows, :] = keep.astype(jnp.int32)
            nbat_ref[rows, :] = jnp.where(keep, bb, B)
            npos_ref[rows, :] = ks + racc
        nbat_ref[pl.ds(NPAD, NXR - NPAD), :] = jnp.full((NXR - NPAD, 1), B,
                                                        jnp.int32)
        nx_ref[pl.ds(0, N), :] = y_ref[...] * scol_ref[pl.ds(0, N), :]

        def one(b, _):
            st = s0_ref[0, b]
            st8 = (st // 8) * 8
            win = nx_ref[pl.ds(st8, RWIN), :]
            bwin = nbat_ref[pl.ds(st8, RWIN), :]
            m = bwin == b
            neg = jnp.float32(-jnp.inf)
            gmp = jnp.max(jnp.where(m, win, neg), axis=0, keepdims=True)
            cb = kf_ref[pl.ds(b, 1), :]
            gap = jnp.sum(jnp.where(m, win, 0.0), axis=0, keepdims=True) / cb
            o_ref[pl.ds(b, 1), 0:64] = gmp
            o_ref[pl.ds(b, 1), 64:128] = gap
            return 0

        lax.fori_loop(0, B, one, 0)

    def run(y, scol, bcol, pcol, spad, bpad, ppad, cnt_row, cnt_col, s0):
        outs = [jax.ShapeDtypeStruct((NXR, 64), jnp.float32),
                jax.ShapeDtypeStruct((NPAD, 1), jnp.int32),
                jax.ShapeDtypeStruct((NXR, 1), jnp.int32),
                jax.ShapeDtypeStruct((NPAD, 1), jnp.int32),
                jax.ShapeDtypeStruct((1, 128), jnp.int32),
                jax.ShapeDtypeStruct((128, 1), jnp.int32),
                jax.ShapeDtypeStruct((B, 128), jnp.float32)]
        specs = [pl.BlockSpec(memory_space=pltpu.VMEM)] * 9 + [
            pl.BlockSpec(memory_space=pltpu.SMEM)]
        return pl.pallas_call(
            body, in_specs=specs, out_shape=outs,
            scratch_shapes=[pltpu.VMEM((128, 1), jnp.float32)],
        )(y, scol, bcol, pcol, spad, bpad, ppad, cnt_row, cnt_col, s0)

    return run


_pool_half = _make_pool(0.5)
_pool_03 = _make_pool(0.3)


def _head_body(x1, x2, x3, x4, w1, b1, w2, b2, w3, b3, o_ref):
    h = x4[...] + x3[...] + x2[...] + x1[...]
    h = jnp.maximum(jnp.dot(h, w1[...], preferred_element_type=jnp.float32)
                    + b1[...], 0.0)
    h = jnp.maximum(jnp.dot(h, w2[...], preferred_element_type=jnp.float32)
                    + b2[...], 0.0)
    z = jnp.dot(h, w3[...], preferred_element_type=jnp.float32) + b3[...]
    zm = z - jnp.max(z, axis=1, keepdims=True)
    o_ref[...] = zm - jnp.log(jnp.sum(jnp.exp(zm), axis=1, keepdims=True))


def _head(x1, x2, x3, x4, p):
    return pl.pallas_call(
        _head_body,
        out_shape=jax.ShapeDtypeStruct((B, 16), jnp.float32),
    )(x1, x2, x3, x4,
      p['l1_W'], p['l1_b'].reshape(1, -1),
      p['l2_W'], p['l2_b'].reshape(1, -1),
      p['l3_W'], p['l3_b'].reshape(1, -1))


def _pad_col(v, rows, fill):
    return jnp.pad(v, ((0, rows - v.shape[0]), (0, 0)), constant_values=fill)


def kernel(x, edge_index, batch, params):
    p = params
    i32 = jnp.int32
    src = jnp.concatenate([edge_index[0].astype(i32),
                           jnp.zeros((EPAD - E,), i32)]).reshape(NW, EC, 128)
    dst = jnp.concatenate([edge_index[1].astype(i32),
                           jnp.full((EPAD - E,), N, i32)]).reshape(NW, EC, 128)
    zero64 = jnp.zeros((NSEG, 64), jnp.float32)
    zero32 = jnp.zeros((NSEG, 32), jnp.float32)
    act_all = jnp.ones((NSEG,), i32)
    ones_mf = jnp.ones((N, 1), jnp.float32)
    dummy_w = jnp.zeros((32, 1), jnp.float32)

    bcol = _pad_col(batch.astype(i32)[:, None], NPAD, B)
    pcol = jnp.arange(NPAD, dtype=i32)[:, None]
    cnt_row, s0 = _seg_setup(bcol)
    cnt_col = cnt_row.reshape(128, 1)

    agg = _agg128_nomask(x, src, dst, act_all, zero64)
    y, _ = _conv_post_first(agg, x, p['c1_Wr'], p['c1_Wroot'],
                            p['c1_b'].reshape(1, -1), p['bn1_g'].reshape(1, -1),
                            p['bn1_b'].reshape(1, -1), ones_mf, dummy_w)
    agg = _agg32_nomask(y, src, dst, act_all, zero32)
    y, scol = _conv_post_score(agg, y, p['c1b_Wr'], p['c1b_Wroot'],
                               p['c1b_b'].reshape(1, -1),
                               p['bn1b_g'].reshape(1, -1),
                               p['bn1b_b'].reshape(1, -1), ones_mf,
                               p['p1_w'].reshape(-1, 1))

    xs = []
    stages = [('c2', 'bn2', 'p2_w'),
              ('c3', 'bn3', 'p3_w'),
              ('c4', 'bn4', 'p4_w'),
              (None, None, None)]
    pools = [_pool_half, _pool_half, _pool_half, _pool_03]
    for li in range(4):
        scp = _pad_col(scol, NPAD, 0.0)
        spad = jnp.pad(scp.reshape(NBLK, 128), ((2, 2), (0, 0)))
        bpad = jnp.pad(bcol.reshape(NBLK, 128), ((2, 2), (0, 0)),
                       constant_values=B)
        ppad = jnp.pad(pcol.reshape(NBLK, 128), ((2, 2), (0, 0)))
        nx, actc, nbat, npos, k_row, k_col, xcat = pools[li](
            y, scp, bcol, pcol, spad, bpad, ppad, cnt_row, cnt_col, s0)
        bcol, pcol, cnt_row, cnt_col = nbat[:NPAD], npos, k_row, k_col
        xs.append(xcat)
        cname, bname, wname = stages[li]
        if cname is None:
            break
        act_ext = jnp.concatenate([actc[:N, 0], jnp.zeros((NSEG - N,), i32)])
        mf = actc[:N].astype(jnp.float32)
        agg = _agg64_mask(nx, src, dst, act_ext, zero64)
        y, scol = _conv_post_score(agg, nx, p[cname + '_Wr'],
                                   p[cname + '_Wroot'],
                                   p[cname + '_b'].reshape(1, -1),
                                   p[bname + '_g'].reshape(1, -1),
                                   p[bname + '_b'].reshape(1, -1), mf,
                                   p[wname].reshape(-1, 1))

    return _head(xs[0], xs[1], xs[2], xs[3], p)


# speedup vs baseline: 1.1240x; 1.0681x over previous
import functools

import jax
import jax.numpy as jnp
from jax import lax
from jax.experimental import pallas as pl
from jax.experimental.pallas import tpu as pltpu
from jax.experimental.pallas import tpu_sc as plsc

N = 10000
E = 320000
B = 128
NW = 32
EC = 80
EPAD = NW * EC * 128
NSEG = 10240
ROWS_PT = NSEG // 16
NPAD = 10112
NBLK = NPAD // 128
RWIN = 264
NXR = 10384
FMAX_SEG = 257


def _make_agg(F, mask_edges, nh=1):
    mesh = plsc.VectorSubcoreMesh(core_axis_name="c", subcore_axis_name="s")

    @functools.partial(
        pl.kernel,
        mesh=mesh,
        compiler_params=pltpu.CompilerParams(use_tc_tiling_on_sc=False,
                                             needs_layout_passes=False),
        out_type=jax.ShapeDtypeStruct((nh, 2, NSEG, F), jnp.float32),
        scratch_types=[
            pltpu.VMEM((EC, 128), jnp.int32),
            pltpu.VMEM((EC, 128), jnp.int32),
            pltpu.VMEM((NSEG,), jnp.int32),
            pltpu.VMEM((4, 128, F), jnp.float32),
            pltpu.VMEM_SHARED((NSEG, F), jnp.float32),
            pltpu.SemaphoreType.DMA,
            pltpu.SemaphoreType.DMA,
            pltpu.SemaphoreType.DMA,
            pltpu.SemaphoreType.DMA,
        ],
    )
    def agg(*refs):
        h_hbms = refs[:nh]
        (src_hbm, dst_hbm, act_hbm, zero_hbm, out_hbm,
         src_v, dst_v, act_v, rows_v, acc, sem0, sem1, sem2, sem3) = refs[nh:]
        c = lax.axis_index("c")
        s = lax.axis_index("s")
        wid = s * 2 + c
        pltpu.sync_copy(src_hbm.at[wid], src_v)
        pltpu.sync_copy(dst_hbm.at[wid], dst_v)
        if mask_edges:
            pltpu.sync_copy(act_hbm, act_v)

            def mask_body(j, _):
                for l in range(8):
                    s16 = src_v[j, pl.ds(l * 16, 16)]
                    d16 = dst_v[j, pl.ds(l * 16, 16)]
                    a_s = plsc.load_gather(act_v, [s16])
                    a_d = plsc.load_gather(act_v, [d16])
                    ok = (a_s + a_d) == 2
                    dst_v[j, pl.ds(l * 16, 16)] = jnp.where(ok, d16, N)
                return 0

            lax.fori_loop(0, EC, mask_body, 0)

        sems = (sem0, sem1, sem2, sem3)
        for ph in range(nh):
            h_hbm = h_hbms[ph]
            pltpu.sync_copy(zero_hbm.at[pl.ds(s * ROWS_PT, ROWS_PT)],
                            acc.at[pl.ds(s * ROWS_PT, ROWS_PT)])
            plsc.subcore_barrier()

            def gstart(j, b):
                pltpu.async_copy(h_hbm.at[src_v.at[j]], rows_v.at[b], sems[b])

            def gwait(b):
                pltpu.make_async_copy(h_hbm.at[src_v.at[0]], rows_v.at[b],
                                      sems[b]).wait()

            def scat(j, b):
                pltpu.sync_copy(rows_v.at[b], acc.at[dst_v.at[j]], add=True)

            for b in range(4):
                gstart(b, b)

            def chunk4(jj, _):
                j0 = jj * 4
                for b in range(4):
                    gwait(b)
                    scat(j0 + b, b)

                    @pl.when(j0 + b + 4 < EC)
                    def _():
                        gstart(j0 + b + 4, b)
                return 0

            lax.fori_loop(0, EC // 4, chunk4, 0)
            plsc.subcore_barrier()
            pltpu.sync_copy(acc.at[pl.ds(s * ROWS_PT, ROWS_PT)],
                            out_hbm.at[ph, c, pl.ds(s * ROWS_PT, ROWS_PT)])

    return agg


@functools.lru_cache(maxsize=None)
def _get_agg(F, mask_edges, nh=1):
    return _make_agg(F, mask_edges, nh)


def _agg128_nomask(h, src, dst, act, zero64):
    f = _get_agg(64, False, 1)
    return jnp.concatenate(
        [f(h[:, :64], src, dst, act, zero64),
         f(h[:, 64:], src, dst, act, zero64)], axis=0)


def _agg32_nomask(*a):
    return _get_agg(32, False)(*a)


def _agg64_mask(*a):
    return _get_agg(64, True)(*a)


def _make_conv_post(needs_proj, with_score, split_agg=False):
    def body(agg_ref, x_ref, wr_ref, wroot_ref, b_ref, g_ref, be_ref,
             mf_ref, w_ref, y_ref, s_ref=None):
        if split_agg:
            a = jnp.concatenate(
                [agg_ref[0, 0, :N, :] + agg_ref[0, 1, :N, :],
                 agg_ref[1, 0, :N, :] + agg_ref[1, 1, :N, :]], axis=1)
        else:
            a = agg_ref[0, 0, :N, :] + agg_ref[0, 1, :N, :]
        if needs_proj:
            a = jnp.dot(a, wr_ref[...], preferred_element_type=jnp.float32)
        pre = a + b_ref[...] + jnp.dot(x_ref[pl.ds(0, N), :], wroot_ref[...],
                                       preferred_element_type=jnp.float32)
        mf = mf_ref[...]
        n = jnp.sum(mf)
        mu = jnp.sum(pre * mf, axis=0, keepdims=True) / n
        var = jnp.sum(((pre - mu) * mf) ** 2, axis=0, keepdims=True) / n
        y = g_ref[...] * (pre - mu) * lax.rsqrt(var + 1e-5) + be_ref[...]
        y = jnp.maximum(y, 0.0)
        y_ref[...] = y
        if with_score:
            w = w_ref[...]
            nw = jnp.sqrt(jnp.sum(w * w))
            s_ref[...] = jnp.tanh(jnp.dot(y, w, preferred_element_type=jnp.float32) / nw)

    def run(agg, x, wr, wroot, bias, g, be, mf, w):
        fout = wroot.shape[1]
        outs = [jax.ShapeDtypeStruct((N, fout), jnp.float32)]
        if with_score:
            outs.append(jax.ShapeDtypeStruct((N, 1), jnp.float32))
        r = pl.pallas_call(body, out_shape=outs)(
            agg, x, wr, wroot, bias, g, be, mf, w)
        return r if with_score else (r[0], None)

    return run


_conv_post_first = _make_conv_post(True, False, split_agg=True)
_conv_post_score = _make_conv_post(True, True)


def _seg_setup_body(b_ref, cnt_ref, s0_ref):
    jj = lax.broadcasted_iota(jnp.int32, (128, 128), 1)
    cnt = jnp.zeros((1, 128), jnp.int32)
    s0 = jnp.zeros((1, 128), jnp.int32)
    for r in range(NBLK):
        bb = b_ref[pl.ds(r * 128, 128), :]
        cnt = cnt + jnp.sum((bb == jj).astype(jnp.int32), axis=0, keepdims=True)
        s0 = s0 + jnp.sum((bb < jj).astype(jnp.int32), axis=0, keepdims=True)
    cnt_ref[...] = cnt
    s0_ref[...] = s0


def _seg_setup(bcol):
    return pl.pallas_call(
        _seg_setup_body,
        out_shape=[jax.ShapeDtypeStruct((1, 128), jnp.int32),
                   jax.ShapeDtypeStruct((1, 128), jnp.int32)],
    )(bcol)


def _make_pool(ratio):
    def body(y_ref, scol_ref, bcol_ref, pcol_ref, spad_ref, bpad_ref,
             ppad_ref, cnt_ref, nx_ref, act_ref, nbat_ref, npos_ref, k_ref):
        kv = jnp.ceil(ratio * cnt_ref[...].astype(jnp.float32)).astype(jnp.int32)
        jj = lax.broadcasted_iota(jnp.int32, (128, 128), 1)
        for r in range(NBLK):
            rows = pl.ds(r * 128, 128)
            bb = bcol_ref[rows, :]
            sb = scol_ref[rows, :]
            pb = pcol_ref[rows, :]
            racc = jnp.zeros((128, 1), jnp.int32)
            for w in range(5):
                sw = spad_ref[pl.ds(r + w, 1), :]
                bw = bpad_ref[pl.ds(r + w, 1), :]
                pw = ppad_ref[pl.ds(r + w, 1), :]
                same = (bw == bb) & (bb < B)
                better = (sw > sb) | ((sw == sb) & (pw < pb))
                racc = racc + jnp.sum((same & better).astype(jnp.int32),
                                      axis=1, keepdims=True)
            kk = jnp.sum(jnp.where(jj == bb, kv, 0), axis=1, keepdims=True)
            ks = jnp.sum(jnp.where(jj < bb, kv, 0), axis=1, keepdims=True)
            keep = (bb < B) & (racc < kk)
            act_ref[rows, :] = keep.astype(jnp.int32)
            nbat_ref[rows, :] = jnp.where(keep, bb, B)
            npos_ref[rows, :] = ks + racc
        nx_ref[...] = y_ref[...] * scol_ref[pl.ds(0, N), :]
        k_ref[...] = kv

    def run(y, scol, bcol, pcol, spad, bpad, ppad, cnt):
        outs = [jax.ShapeDtypeStruct((N, 64), jnp.float32),
                jax.ShapeDtypeStruct((NPAD, 1), jnp.int32),
                jax.ShapeDtypeStruct((NPAD, 1), jnp.int32),
                jax.ShapeDtypeStruct((NPAD, 1), jnp.int32),
                jax.ShapeDtypeStruct((1, 128), jnp.int32)]
        return pl.pallas_call(body, out_shape=outs)(
            y, scol, bcol, pcol, spad, bpad, ppad, cnt)

    return run


_pool_half = _make_pool(0.5)
_pool_03 = _make_pool(0.3)


def _readout_body(x_ref, bcol_ref, s0_ref, cnt_ref, o_ref):
    def one(b, _):
        st = s0_ref[0, b]
        st8 = (st // 8) * 8
        win = x_ref[pl.ds(st8, RWIN), :]
        bwin = bcol_ref[pl.ds(st8, RWIN), :]
        m = bwin == b
        neg = jnp.float32(-jnp.inf)
        gmp = jnp.max(jnp.where(m, win, neg), axis=0, keepdims=True)
        cb = jnp.maximum(cnt_ref[0, b].astype(jnp.float32), 1.0)
        gap = jnp.sum(jnp.where(m, win, 0.0), axis=0, keepdims=True) / cb
        o_ref[pl.ds(b, 1), 0:64] = gmp
        o_ref[pl.ds(b, 1), 64:128] = gap
        return 0

    lax.fori_loop(0, B, one, 0)


def _readout(x_pad, bcol_pad, s0, cnt):
    return pl.pallas_call(
        _readout_body,
        in_specs=[pl.BlockSpec(memory_space=pltpu.VMEM),
                  pl.BlockSpec(memory_space=pltpu.VMEM),
                  pl.BlockSpec(memory_space=pltpu.SMEM),
                  pl.BlockSpec(memory_space=pltpu.SMEM)],
        out_shape=jax.ShapeDtypeStruct((B, 128), jnp.float32),
    )(x_pad, bcol_pad, s0, cnt)


def _head_body(x1, x2, x3, x4, w1, b1, w2, b2, w3, b3, o_ref):
    h = x4[...] + x3[...] + x2[...] + x1[...]
    h = jnp.maximum(jnp.dot(h, w1[...], preferred_element_type=jnp.float32)
                    + b1[...], 0.0)
    h = jnp.maximum(jnp.dot(h, w2[...], preferred_element_type=jnp.float32)
                    + b2[...], 0.0)
    z = jnp.dot(h, w3[...], preferred_element_type=jnp.float32) + b3[...]
    zm = z - jnp.max(z, axis=1, keepdims=True)
    o_ref[...] = zm - jnp.log(jnp.sum(jnp.exp(zm), axis=1, keepdims=True))


def _head(x1, x2, x3, x4, p):
    return pl.pallas_call(
        _head_body,
        out_shape=jax.ShapeDtypeStruct((B, 16), jnp.float32),
    )(x1, x2, x3, x4,
      p['l1_W'], p['l1_b'].reshape(1, -1),
      p['l2_W'], p['l2_b'].reshape(1, -1),
      p['l3_W'], p['l3_b'].reshape(1, -1))


def _pad_col(v, rows, fill):
    return jnp.pad(v, ((0, rows - v.shape[0]), (0, 0)), constant_values=fill)


def kernel(x, edge_index, batch, params):
    p = params
    i32 = jnp.int32
    src = jnp.concatenate([edge_index[0].astype(i32),
                           jnp.zeros((EPAD - E,), i32)]).reshape(NW, EC, 128)
    dst = jnp.concatenate([edge_index[1].astype(i32),
                           jnp.full((EPAD - E,), N, i32)]).reshape(NW, EC, 128)
    zero64 = jnp.zeros((NSEG, 64), jnp.float32)
    zero32 = jnp.zeros((NSEG, 32), jnp.float32)
    act_all = jnp.ones((NSEG,), i32)
    ones_mf = jnp.ones((N, 1), jnp.float32)
    dummy_w = jnp.zeros((32, 1), jnp.float32)

    bcol = _pad_col(batch.astype(i32)[:, None], NPAD, B)
    pcol = jnp.arange(NPAD, dtype=i32)[:, None]
    cnt, s0 = _seg_setup(bcol)

    agg = _agg128_nomask(x, src, dst, act_all, zero64)
    y, _ = _conv_post_first(agg, x, p['c1_Wr'], p['c1_Wroot'],
                            p['c1_b'].reshape(1, -1), p['bn1_g'].reshape(1, -1),
                            p['bn1_b'].reshape(1, -1), ones_mf, dummy_w)
    agg = _agg32_nomask(y, src, dst, act_all, zero32)
    y, scol = _conv_post_score(agg, y, p['c1b_Wr'], p['c1b_Wroot'],
                               p['c1b_b'].reshape(1, -1),
                               p['bn1b_g'].reshape(1, -1),
                               p['bn1b_b'].reshape(1, -1), ones_mf,
                               p['p1_w'].reshape(-1, 1))

    xs = []
    stages = [('c2', 'bn2', 'p2_w'),
              ('c3', 'bn3', 'p3_w'),
              ('c4', 'bn4', 'p4_w'),
              (None, None, None)]
    pools = [_pool_half, _pool_half, _pool_half, _pool_03]
    for li in range(4):
        scp = _pad_col(scol, NPAD, 0.0)
        spad = jnp.pad(scp.reshape(NBLK, 128), ((2, 2), (0, 0)))
        bpad = jnp.pad(bcol.reshape(NBLK, 128), ((2, 2), (0, 0)),
                       constant_values=B)
        ppad = jnp.pad(pcol.reshape(NBLK, 128), ((2, 2), (0, 0)))
        nx, actc, nbat, npos, k = pools[li](y, scp, bcol, pcol,
                                            spad, bpad, ppad, cnt)
        bcol, pcol, cnt = nbat, npos, k
        x_pad = jnp.pad(nx, ((0, NXR - N), (0, 0)))
        b_pad = _pad_col(nbat, NXR, B)
        xs.append(_readout(x_pad, b_pad, s0, cnt))
        cname, bname, wname = stages[li]
        if cname is None:
            break
        act_ext = jnp.concatenate([actc[:N, 0], jnp.zeros((NSEG - N,), i32)])
        mf = actc[:N].astype(jnp.float32)
        agg = _agg64_mask(nx, src, dst, act_ext, zero64)
        y, scol = _conv_post_score(agg, nx, p[cname + '_Wr'],
                                   p[cname + '_Wroot'],
                                   p[cname + '_b'].reshape(1, -1),
                                   p[bname + '_g'].reshape(1, -1),
                                   p[bname + '_b'].reshape(1, -1), mf,
                                   p[wname].reshape(-1, 1))

    return _head(xs[0], xs[1], xs[2], xs[3], p)
